# Initial kernel scaffold; baseline (speedup 1.0000x reference)
import jax
import jax.numpy as jnp
from jax.experimental import pallas as pl


def kernel(x, rbf, sbf, idx_kj, idx_ji, w):
    raise NotImplementedError("write your pallas kernel here")


# speedup vs baseline: 2.5517x; 2.5517x over previous
import functools

import jax
import jax.numpy as jnp
from jax import lax
from jax.experimental import pallas as pl
from jax.experimental.pallas import tpu as pltpu
from jax.experimental.pallas import tpu_sc as plsc

E = 160000
T = 1280000
H = 128
INT = 64
BAS = 8
NR = 6
NSR = 42

NC = 2
NS = 16
L = 16

NG = 8
GW = 8
GPC = NG // NC

W = 1000
TPS = T // NS
NW = TPS // W
EPS = E // NS
ZR = 1000

EB = 2000
TB = 2560


def _swish(v):
    return v * jax.nn.sigmoid(v)


def _stage_a_body(x_ref, rbf_ref, wji, bji, wkj, bkj, wr1, wr2, wdn,
                  xji_out, *xg_outs):
    xb = x_ref[...]
    xji = _swish(xb @ wji[...] + bji[...])
    xkj = _swish(xb @ wkj[...] + bkj[...])
    rbf_e = (rbf_ref[...] @ wr1[...]) @ wr2[...]
    xkj = xkj * rbf_e
    xkj64 = _swish(xkj @ wdn[...])
    xji_out[...] = xji
    for g in range(NG):
        xg_outs[g][...] = xkj64[:, g * GW:(g + 1) * GW]


def _stage_a(x, rbf, w):
    full = lambda shp: pl.BlockSpec(shp, lambda i: (0, 0))
    out_shape = ([jax.ShapeDtypeStruct((E, H), jnp.float32)]
                 + [jax.ShapeDtypeStruct((E, GW), jnp.float32)] * NG)
    out_specs = ([pl.BlockSpec((EB, H), lambda i: (i, 0))]
                 + [pl.BlockSpec((EB, GW), lambda i: (i, 0))] * NG)
    return pl.pallas_call(
        _stage_a_body,
        grid=(E // EB,),
        in_specs=[pl.BlockSpec((EB, H), lambda i: (i, 0)),
                  pl.BlockSpec((EB, NR), lambda i: (i, 0)),
                  full((H, H)), full((1, H)), full((H, H)), full((1, H)),
                  full((NR, BAS)), full((BAS, H)), full((H, INT))],
        out_specs=out_specs,
        out_shape=out_shape,
    )(x, rbf, w["W_ji"], w["b_ji"].reshape(1, H),
      w["W_kj"], w["b_kj"].reshape(1, H),
      w["W_rbf1"], w["W_rbf2"], w["W_down"])


def _stage_b_body(sbf_ref, ws1, ws2, *sg_outs):
    se = (sbf_ref[...] @ ws1[...]) @ ws2[...]
    for g in range(NG):
        sg_outs[g][...] = se[:, g * GW:(g + 1) * GW]


def _stage_b(sbf, w):
    full = lambda shp: pl.BlockSpec(shp, lambda i: (0, 0))
    return pl.pallas_call(
        _stage_b_body,
        grid=(T // TB,),
        in_specs=[pl.BlockSpec((TB, NSR), lambda i: (i, 0)),
                  full((NSR, BAS)), full((BAS, INT))],
        out_specs=[pl.BlockSpec((TB, GW), lambda i: (i, 0))] * NG,
        out_shape=[jax.ShapeDtypeStruct((T, GW), jnp.float32)] * NG,
    )(sbf, w["W_sbf1"], w["W_sbf2"])


def _stage_c(xgs, sgs, idx_kj, idx_ji, zrows):
    mesh = plsc.VectorSubcoreMesh(core_axis_name="c", subcore_axis_name="s")
    out_type = [jax.ShapeDtypeStruct((E, GW), jnp.float32) for _ in range(NG)]
    scratch_types = [
        pltpu.VMEM((W,), jnp.int32),
        pltpu.VMEM((W,), jnp.int32),
        pltpu.VMEM((W, GW), jnp.float32),
        pltpu.VMEM((W, GW), jnp.float32),
        pltpu.VMEM((W, GW), jnp.float32),
        pltpu.VMEM((ZR, GW), jnp.float32),
        pltpu.VMEM_SHARED((E, GW), jnp.float32),
        pltpu.SemaphoreType.DMA,
    ]

    @functools.partial(pl.kernel, mesh=mesh, out_type=out_type,
                       scratch_types=scratch_types,
                       compiler_params=pltpu.CompilerParams(
                           needs_layout_passes=False,
                           use_tc_tiling_on_sc=False))
    def c_kernel(*refs):
        xg = refs[0:NG]
        sg = refs[NG:2 * NG]
        kj_hbm = refs[2 * NG]
        ji_hbm = refs[2 * NG + 1]
        z_hbm = refs[2 * NG + 2]
        outs = refs[2 * NG + 3:2 * NG + 3 + NG]
        kjb, jib, rb, sb, mb, zb, table, sem = refs[2 * NG + 3 + NG:]

        cid = lax.axis_index("c")
        sid = lax.axis_index("s")
        base_t = sid * TPS
        base_e = sid * EPS

        pltpu.sync_copy(z_hbm, zb)

        lane = lax.iota(jnp.int32, L)
        rsub = jnp.right_shift(lane, 3)
        csub = jnp.bitwise_and(lane, GW - 1)

        def group_pass(g):
            @pl.when(cid == (g // GPC))
            def _():
                for k in range(EPS // ZR):
                    pltpu.sync_copy(zb, table.at[pl.ds(base_e + k * ZR, ZR)])
                plsc.subcore_barrier()

                def win(wi, carry):
                    off = base_t + wi * W
                    pltpu.sync_copy(kj_hbm.at[pl.ds(off, W)], kjb)
                    pltpu.sync_copy(ji_hbm.at[pl.ds(off, W)], jib)
                    pltpu.sync_copy(sg[g].at[pl.ds(off, W)], sb)
                    pltpu.async_copy(xg[g].at[kjb], rb, sem).wait()

                    def mul(i, r):
                        gv = plsc.load_gather(rb, [r, csub])
                        sv = plsc.load_gather(sb, [r, csub])
                        plsc.store_scatter(mb, [r, csub], gv * sv)
                        return r + (L // GW)

                    lax.fori_loop(0, W * GW // L, mul, rsub)
                    pltpu.sync_copy(mb, table.at[jib], add=True)
                    return carry

                lax.fori_loop(0, NW, win, 0)
                plsc.subcore_barrier()
                pltpu.sync_copy(table.at[pl.ds(base_e, EPS)],
                                outs[g].at[pl.ds(base_e, EPS)])
                plsc.subcore_barrier()

        for g in range(NG):
            group_pass(g)

    return c_kernel(*xgs, *sgs, idx_kj, idx_ji, zrows)


def _stage_d_body(xji_ref, x_ref, a0, a1, a2, a3, a4, a5, a6, a7,
                  wup, wb00, bb00, wb01, bb01, wsk, bsk,
                  wa00, ba00, wa01, ba01, wa10, ba10, wa11, ba11, out_ref):
    agg = jnp.concatenate([a[...] for a in (a0, a1, a2, a3, a4, a5, a6, a7)],
                          axis=1)
    xkj = _swish(agg @ wup[...])
    h = xji_ref[...] + xkj
    h = h + _swish(_swish(h @ wb00[...] + bb00[...]) @ wb01[...] + bb01[...])
    h = _swish(h @ wsk[...] + bsk[...]) + x_ref[...]
    h = h + _swish(_swish(h @ wa00[...] + ba00[...]) @ wa01[...] + ba01[...])
    h = h + _swish(_swish(h @ wa10[...] + ba10[...]) @ wa11[...] + ba11[...])
    out_ref[...] = h


def _stage_d(xji, x, ags, w):
    full = lambda shp: pl.BlockSpec(shp, lambda i: (0, 0))
    b = lambda name: w[name].reshape(1, H)
    return pl.pallas_call(
        _stage_d_body,
        grid=(E // EB,),
        in_specs=([pl.BlockSpec((EB, H), lambda i: (i, 0))] * 2
                  + [pl.BlockSpec((EB, GW), lambda i: (i, 0))] * NG
                  + [full((INT, H)),
                     full((H, H)), full((1, H)), full((H, H)), full((1, H)),
                     full((H, H)), full((1, H)),
                     full((H, H)), full((1, H)), full((H, H)), full((1, H)),
                     full((H, H)), full((1, H)), full((H, H)), full((1, H))]),
        out_specs=pl.BlockSpec((EB, H), lambda i: (i, 0)),
        out_shape=jax.ShapeDtypeStruct((E, H), jnp.float32),
    )(xji, x, *ags,
      w["W_up"], w["Wb0_0"], b("bb0_0"), w["Wb0_1"], b("bb0_1"),
      w["W_skip"], b("b_skip"),
      w["Wa0_0"], b("ba0_0"), w["Wa0_1"], b("ba0_1"),
      w["Wa1_0"], b("ba1_0"), w["Wa1_1"], b("ba1_1"))


def kernel(x, rbf, sbf, idx_kj, idx_ji, w):
    idx_kj = idx_kj.astype(jnp.int32)
    idx_ji = idx_ji.astype(jnp.int32)
    outs_a = _stage_a(x, rbf, w)
    xji, xgs = outs_a[0], outs_a[1:]
    sgs = _stage_b(sbf, w)
    zrows = jnp.zeros((ZR, GW), jnp.float32)
    ags = _stage_c(xgs, sgs, idx_kj, idx_ji, zrows)
    return _stage_d(xji, x, ags, w)


# speedup vs baseline: 4.8289x; 1.8924x over previous
import functools

import jax
import jax.numpy as jnp
from jax import lax
from jax.experimental import pallas as pl
from jax.experimental.pallas import tpu as pltpu
from jax.experimental.pallas import tpu_sc as plsc

E = 160000
T = 1280000
H = 128
INT = 64
BAS = 8
NR = 6
NSR = 42

NC = 2
NS = 16
L = 16

NG = 8
GW = 8
GPC = NG // NC

TH = T // 2
EH = E // 2
W = 800
PPS = TH // NS
NW = PPS // W
EPS = E // NS
ZR = 500

EB = 2000
EB2 = 4000
TBB = 6400


def _swish(v):
    return v * jax.nn.sigmoid(v)


def _stage_a_body(x_ref, rbf_ref, wji, bji, wkj, bkj, wr12, wdn,
                  xji_out, xkj_out):
    xb = x_ref[...]
    xji = _swish(xb @ wji[...] + bji[...])
    xkj = _swish(xb @ wkj[...] + bkj[...])
    rbf_e = rbf_ref[...] @ wr12[...]
    xkj = xkj * rbf_e
    xkj_out[...] = _swish(xkj @ wdn[...])
    xji_out[...] = xji


def _stage_a(x, rbf, w):
    full = lambda shp: pl.BlockSpec(shp, lambda i: (0, 0))
    wr12 = w["W_rbf1"] @ w["W_rbf2"]
    return pl.pallas_call(
        _stage_a_body,
        grid=(E // EB,),
        in_specs=[pl.BlockSpec((EB, H), lambda i: (i, 0)),
                  pl.BlockSpec((EB, NR), lambda i: (i, 0)),
                  full((H, H)), full((1, H)), full((H, H)), full((1, H)),
                  full((NR, H)), full((H, INT))],
        out_specs=[pl.BlockSpec((EB, H), lambda i: (i, 0)),
                   pl.BlockSpec((EB, INT), lambda i: (i, 0))],
        out_shape=[jax.ShapeDtypeStruct((E, H), jnp.float32),
                   jax.ShapeDtypeStruct((E, INT), jnp.float32)],
    )(x, rbf, w["W_ji"], w["b_ji"].reshape(1, H),
      w["W_kj"], w["b_kj"].reshape(1, H), wr12, w["W_down"])


def _splitter(xkj64, g):
    def body(xk_ref, out_ref):
        out_ref[...] = xk_ref[:, g * GW:(g + 1) * GW]
    return pl.pallas_call(
        body,
        grid=(E // EB2,),
        in_specs=[pl.BlockSpec((EB2, INT), lambda i: (i, 0))],
        out_specs=pl.BlockSpec((EB2, GW), lambda i: (i, 0)),
        out_shape=jax.ShapeDtypeStruct((E, GW), jnp.float32),
    )(xkj64)


def _stage_b(sbf, w):
    w12 = w["W_sbf1"] @ w["W_sbf2"]
    def body(lo_ref, hi_ref, w12_ref, out_ref):
        lo = lo_ref[...] @ w12_ref[...]
        hi = hi_ref[...] @ w12_ref[...]
        out_ref[...] = jnp.concatenate([lo, hi], axis=1)
    nb = TH // TBB
    return pl.pallas_call(
        body,
        grid=(nb,),
        in_specs=[pl.BlockSpec((TBB, NSR), lambda i: (i, 0)),
                  pl.BlockSpec((TBB, NSR), lambda i, nb=nb: (i + nb, 0)),
                  pl.BlockSpec((NSR, INT), lambda i: (0, 0))],
        out_specs=pl.BlockSpec((TBB, 2 * INT), lambda i: (i, 0)),
        out_shape=jax.ShapeDtypeStruct((TH, 2 * INT), jnp.float32),
    )(sbf, sbf, w12)


def _stage_c(xgs, se_pair, idx_kj, idx_ji, zrows):
    mesh = plsc.VectorSubcoreMesh(core_axis_name="c", subcore_axis_name="s")
    out_type = jax.ShapeDtypeStruct((EH, 4 * INT), jnp.float32)
    scratch_types = [
        pltpu.VMEM((2 * W,), jnp.int32),
        pltpu.VMEM((2 * W,), jnp.int32),
        pltpu.VMEM((2 * W, GW), jnp.float32),
        pltpu.VMEM((2 * W, GW), jnp.float32),
        pltpu.VMEM((2 * W, GW), jnp.float32),
        pltpu.VMEM((ZR, GW), jnp.float32),
        pltpu.VMEM_SHARED((E, GW), jnp.float32),
        pltpu.SemaphoreType.DMA,
    ]

    @functools.partial(pl.kernel, mesh=mesh, out_type=out_type,
                       scratch_types=scratch_types,
                       compiler_params=pltpu.CompilerParams(
                           needs_layout_passes=False,
                           use_tc_tiling_on_sc=False))
    def c_kernel(*refs):
        xg = refs[0:NG]
        se_hbm = refs[NG]
        kj_hbm = refs[NG + 1]
        ji_hbm = refs[NG + 2]
        z_hbm = refs[NG + 3]
        out_hbm = refs[NG + 4]
        kjb, jib, rb, sb, mb, zb, table, sem = refs[NG + 5:]

        cid = lax.axis_index("c")
        sid = lax.axis_index("s")
        base_p = sid * PPS
        base_e = sid * EPS
        out_row = jnp.where(sid < NS // 2, base_e, base_e - EH)
        out_colsel = jnp.where(sid < NS // 2, 0, 2 * INT)

        pltpu.sync_copy(z_hbm, zb)

        lane = lax.iota(jnp.int32, L)
        rsub = jnp.right_shift(lane, 3)
        csub = jnp.bitwise_and(lane, GW - 1)

        def group_pass(g):
            @pl.when(cid == (g // GPC))
            def _():
                for k in range(EPS // ZR):
                    pltpu.sync_copy(zb, table.at[pl.ds(base_e + k * ZR, ZR)])
                plsc.subcore_barrier()

                def win(wi, carry):
                    off = base_p + wi * W
                    pltpu.sync_copy(kj_hbm.at[pl.ds(off, W)],
                                    kjb.at[pl.ds(0, W)])
                    pltpu.sync_copy(kj_hbm.at[pl.ds(TH + off, W)],
                                    kjb.at[pl.ds(W, W)])
                    pltpu.sync_copy(ji_hbm.at[pl.ds(off, W)],
                                    jib.at[pl.ds(0, W)])
                    pltpu.sync_copy(ji_hbm.at[pl.ds(TH + off, W)],
                                    jib.at[pl.ds(W, W)])
                    pltpu.sync_copy(
                        se_hbm.at[pl.ds(off, W), pl.ds(g * GW, GW)],
                        sb.at[pl.ds(0, W)])
                    pltpu.sync_copy(
                        se_hbm.at[pl.ds(off, W), pl.ds(INT + g * GW, GW)],
                        sb.at[pl.ds(W, W)])
                    pltpu.async_copy(xg[g].at[kjb], rb, sem).wait()

                    def mul(i, tvec):
                        rv = plsc.load_gather(rb, [tvec, csub])
                        sv = plsc.load_gather(sb, [tvec, csub])
                        plsc.store_scatter(mb, [tvec, csub], rv * sv)
                        return tvec + (L // GW)

                    lax.fori_loop(0, 2 * W * GW // L, mul, rsub, unroll=5)
                    pltpu.sync_copy(mb, table.at[jib], add=True)
                    return carry

                lax.fori_loop(0, NW, win, 0)
                plsc.subcore_barrier()
                pltpu.sync_copy(
                    table.at[pl.ds(base_e, EPS)],
                    out_hbm.at[pl.ds(out_row, EPS),
                               pl.ds(out_colsel + g * GW, GW)])
                plsc.subcore_barrier()

        for g in range(NG):
            group_pass(g)

    return c_kernel(*xgs, se_pair, idx_kj, idx_ji, zrows)


def _stage_d_body(xji_ref, x_ref, agg_ref,
                  wup, wb00, bb00, wb01, bb01, wsk, bsk,
                  wa00, ba00, wa01, ba01, wa10, ba10, wa11, ba11, out_ref):
    agg = agg_ref[...][:, :INT]
    xkj = _swish(agg @ wup[...])
    h = xji_ref[...] + xkj
    h = h + _swish(_swish(h @ wb00[...] + bb00[...]) @ wb01[...] + bb01[...])
    h = _swish(h @ wsk[...] + bsk[...]) + x_ref[...]
    h = h + _swish(_swish(h @ wa00[...] + ba00[...]) @ wa01[...] + ba01[...])
    h = h + _swish(_swish(h @ wa10[...] + ba10[...]) @ wa11[...] + ba11[...])
    out_ref[...] = h


def _stage_d(xji, x, agg_pair, w):
    full = lambda shp: pl.BlockSpec(shp, lambda p, i: (0, 0))
    b = lambda name: w[name].reshape(1, H)
    nbe = EH // EB
    half = lambda p, i, nbe=nbe: (p * nbe + i, 0)
    return pl.pallas_call(
        _stage_d_body,
        grid=(2, nbe),
        in_specs=([pl.BlockSpec((EB, H), half),
                   pl.BlockSpec((EB, H), half),
                   pl.BlockSpec((EB, 2 * INT), lambda p, i: (i, p))]
                  + [full((INT, H)),
                     full((H, H)), full((1, H)), full((H, H)), full((1, H)),
                     full((H, H)), full((1, H)),
                     full((H, H)), full((1, H)), full((H, H)), full((1, H)),
                     full((H, H)), full((1, H)), full((H, H)), full((1, H))]),
        out_specs=pl.BlockSpec((EB, H), half),
        out_shape=jax.ShapeDtypeStruct((E, H), jnp.float32),
    )(xji, x, agg_pair,
      w["W_up"], w["Wb0_0"], b("bb0_0"), w["Wb0_1"], b("bb0_1"),
      w["W_skip"], b("b_skip"),
      w["Wa0_0"], b("ba0_0"), w["Wa0_1"], b("ba0_1"),
      w["Wa1_0"], b("ba1_0"), w["Wa1_1"], b("ba1_1"))


def kernel(x, rbf, sbf, idx_kj, idx_ji, w):
    idx_kj = idx_kj.astype(jnp.int32)
    idx_ji = idx_ji.astype(jnp.int32)
    xji, xkj64 = _stage_a(x, rbf, w)
    xgs = [_splitter(xkj64, g) for g in range(NG)]
    se_pair = _stage_b(sbf, w)
    zrows = jnp.zeros((ZR, GW), jnp.float32)
    agg_pair = _stage_c(xgs, se_pair, idx_kj, idx_ji, zrows)
    return _stage_d(xji, x, agg_pair, w)


# speedup vs baseline: 5.4352x; 1.1256x over previous
import functools

import jax
import jax.numpy as jnp
from jax import lax
from jax.experimental import pallas as pl
from jax.experimental.pallas import tpu as pltpu
from jax.experimental.pallas import tpu_sc as plsc

E = 160000
T = 1280000
H = 128
INT = 64
BAS = 8
NR = 6
NSR = 42

NC = 2
NS = 16
L = 16

NG = 8
GW = 8
GPC = NG // NC

TH = T // 2
EH = E // 2
W = 800
PPS = TH // NS
NW = PPS // W
EPS = E // NS
ZR = 500

EB = 2000
EB2 = 2000
TBB = 6400


def _swish(v):
    return v * jax.nn.sigmoid(v)


def _stage_a_body(x_ref, rbf_ref, wji, bji, wkj, bkj, wr12, wdn,
                  xji_out, xkj_out):
    xb = x_ref[...]
    xji = _swish(xb @ wji[...] + bji[...])
    xkj = _swish(xb @ wkj[...] + bkj[...])
    rbf_e = rbf_ref[...] @ wr12[...]
    xkj = xkj * rbf_e
    xkj_out[...] = _swish(xkj @ wdn[...])
    xji_out[...] = xji


def _stage_a(x, rbf, w):
    full = lambda shp: pl.BlockSpec(shp, lambda i: (0, 0))
    wr12 = w["W_rbf1"] @ w["W_rbf2"]
    return pl.pallas_call(
        _stage_a_body,
        grid=(E // EB,),
        in_specs=[pl.BlockSpec((EB, H), lambda i: (i, 0)),
                  pl.BlockSpec((EB, NR), lambda i: (i, 0)),
                  full((H, H)), full((1, H)), full((H, H)), full((1, H)),
                  full((NR, H)), full((H, INT))],
        out_specs=[pl.BlockSpec((EB, H), lambda i: (i, 0)),
                   pl.BlockSpec((EB, INT), lambda i: (i, 0))],
        out_shape=[jax.ShapeDtypeStruct((E, H), jnp.float32),
                   jax.ShapeDtypeStruct((E, INT), jnp.float32)],
    )(x, rbf, w["W_ji"], w["b_ji"].reshape(1, H),
      w["W_kj"], w["b_kj"].reshape(1, H), wr12, w["W_down"])


def _splitter(xkj64):
    def body(xk_ref, *out_refs):
        xk = xk_ref[...]
        for g in range(NG):
            out_refs[g][...] = xk[:, g * GW:(g + 1) * GW]
    return pl.pallas_call(
        body,
        grid=(E // EB2,),
        in_specs=[pl.BlockSpec((EB2, INT), lambda i: (i, 0))],
        out_specs=[pl.BlockSpec((EB2, GW), lambda i: (i, 0))] * NG,
        out_shape=[jax.ShapeDtypeStruct((E, GW), jnp.float32)] * NG,
    )(xkj64)


def _stage_b(sbf, w):
    import numpy as np
    perm = np.zeros((2 * INT, 2 * INT), np.float32)
    for g in range(NG):
        for c in range(GW):
            perm[g * GW + c, 2 * g * GW + c] = 1.0
            perm[INT + g * GW + c, 2 * g * GW + GW + c] = 1.0
    w12 = w["W_sbf1"] @ w["W_sbf2"]
    w12lo = w12 @ jnp.asarray(perm[:INT])
    w12hi = w12 @ jnp.asarray(perm[INT:])

    def body(lo_ref, hi_ref, wlo_ref, whi_ref, out_ref):
        out_ref[...] = lo_ref[...] @ wlo_ref[...] + hi_ref[...] @ whi_ref[...]
    nb = TH // TBB
    return pl.pallas_call(
        body,
        grid=(nb,),
        in_specs=[pl.BlockSpec((TBB, NSR), lambda i: (i, 0)),
                  pl.BlockSpec((TBB, NSR), lambda i, nb=nb: (i + nb, 0)),
                  pl.BlockSpec((NSR, 2 * INT), lambda i: (0, 0)),
                  pl.BlockSpec((NSR, 2 * INT), lambda i: (0, 0))],
        out_specs=pl.BlockSpec((TBB, 2 * INT), lambda i: (i, 0)),
        out_shape=jax.ShapeDtypeStruct((TH, 2 * INT), jnp.float32),
    )(sbf, sbf, w12lo, w12hi)


def _stage_c(xgs, se_pair, idx_kj, idx_ji, zrows):
    mesh = plsc.VectorSubcoreMesh(core_axis_name="c", subcore_axis_name="s")
    out_type = jax.ShapeDtypeStruct((EH, 4 * INT), jnp.float32)
    scratch_types = [
        pltpu.VMEM((2 * W,), jnp.int32),
        pltpu.VMEM((2 * W,), jnp.int32),
        pltpu.VMEM((2 * W, GW), jnp.float32),
        pltpu.VMEM((W, 2 * GW), jnp.float32),
        pltpu.VMEM((2 * W, GW), jnp.float32),
        pltpu.VMEM((ZR, GW), jnp.float32),
        pltpu.VMEM_SHARED((E, GW), jnp.float32),
        pltpu.SemaphoreType.DMA,
    ]

    @functools.partial(pl.kernel, mesh=mesh, out_type=out_type,
                       scratch_types=scratch_types,
                       compiler_params=pltpu.CompilerParams(
                           needs_layout_passes=False,
                           use_tc_tiling_on_sc=False))
    def c_kernel(*refs):
        xg = refs[0:NG]
        se_hbm = refs[NG]
        kj_hbm = refs[NG + 1]
        ji_hbm = refs[NG + 2]
        z_hbm = refs[NG + 3]
        out_hbm = refs[NG + 4]
        kjb, jib, rb, sb, mb, zb, table, sem = refs[NG + 5:]

        cid = lax.axis_index("c")
        sid = lax.axis_index("s")
        base_p = sid * PPS
        base_e = sid * EPS
        out_row = jnp.where(sid < NS // 2, base_e, base_e - EH)
        out_colsel = jnp.where(sid < NS // 2, 0, 2 * INT)

        pltpu.sync_copy(z_hbm, zb)

        lane = lax.iota(jnp.int32, L)
        rsub = jnp.right_shift(lane, 3)
        csub = jnp.bitwise_and(lane, GW - 1)

        def group_pass(g):
            @pl.when(cid == (g // GPC))
            def _():
                for k in range(EPS // ZR):
                    pltpu.sync_copy(zb, table.at[pl.ds(base_e + k * ZR, ZR)])
                plsc.subcore_barrier()

                def win(wi, carry):
                    off = base_p + wi * W
                    pltpu.sync_copy(kj_hbm.at[pl.ds(off, W)],
                                    kjb.at[pl.ds(0, W)])
                    pltpu.sync_copy(kj_hbm.at[pl.ds(TH + off, W)],
                                    kjb.at[pl.ds(W, W)])
                    pltpu.sync_copy(ji_hbm.at[pl.ds(off, W)],
                                    jib.at[pl.ds(0, W)])
                    pltpu.sync_copy(ji_hbm.at[pl.ds(TH + off, W)],
                                    jib.at[pl.ds(W, W)])
                    pltpu.sync_copy(
                        se_hbm.at[pl.ds(off, W), pl.ds(g * 2 * GW, 2 * GW)],
                        sb)
                    pltpu.async_copy(xg[g].at[kjb], rb, sem).wait()

                    def mul_lo(i, tvec):
                        rv = plsc.load_gather(rb, [tvec, csub])
                        sv = plsc.load_gather(sb, [tvec, csub])
                        plsc.store_scatter(mb, [tvec, csub], rv * sv)
                        return tvec + (L // GW)

                    def mul_hi(i, tvec):
                        rv = plsc.load_gather(rb, [tvec + W, csub])
                        sv = plsc.load_gather(sb, [tvec, csub + GW])
                        plsc.store_scatter(mb, [tvec + W, csub], rv * sv)
                        return tvec + (L // GW)

                    lax.fori_loop(0, W * GW // L, mul_lo, rsub, unroll=5)
                    lax.fori_loop(0, W * GW // L, mul_hi, rsub, unroll=5)
                    pltpu.sync_copy(mb, table.at[jib], add=True)
                    return carry

                lax.fori_loop(0, NW, win, 0)
                plsc.subcore_barrier()
                pltpu.sync_copy(
                    table.at[pl.ds(base_e, EPS)],
                    out_hbm.at[pl.ds(out_row, EPS),
                               pl.ds(out_colsel + g * GW, GW)])
                plsc.subcore_barrier()

        for g in range(NG):
            group_pass(g)

    return c_kernel(*xgs, se_pair, idx_kj, idx_ji, zrows)


def _stage_d_body(xji_ref, x_ref, agg_ref,
                  wup, wb00, bb00, wb01, bb01, wsk, bsk,
                  wa00, ba00, wa01, ba01, wa10, ba10, wa11, ba11, out_ref):
    agg = agg_ref[...][:, :INT]
    xkj = _swish(agg @ wup[...])
    h = xji_ref[...] + xkj
    h = h + _swish(_swish(h @ wb00[...] + bb00[...]) @ wb01[...] + bb01[...])
    h = _swish(h @ wsk[...] + bsk[...]) + x_ref[...]
    h = h + _swish(_swish(h @ wa00[...] + ba00[...]) @ wa01[...] + ba01[...])
    h = h + _swish(_swish(h @ wa10[...] + ba10[...]) @ wa11[...] + ba11[...])
    out_ref[...] = h


def _stage_d(xji, x, agg_pair, w):
    full = lambda shp: pl.BlockSpec(shp, lambda p, i: (0, 0))
    b = lambda name: w[name].reshape(1, H)
    nbe = EH // EB
    half = lambda p, i, nbe=nbe: (p * nbe + i, 0)
    return pl.pallas_call(
        _stage_d_body,
        grid=(2, nbe),
        in_specs=([pl.BlockSpec((EB, H), half),
                   pl.BlockSpec((EB, H), half),
                   pl.BlockSpec((EB, 2 * INT), lambda p, i: (i, p))]
                  + [full((INT, H)),
                     full((H, H)), full((1, H)), full((H, H)), full((1, H)),
                     full((H, H)), full((1, H)),
                     full((H, H)), full((1, H)), full((H, H)), full((1, H)),
                     full((H, H)), full((1, H)), full((H, H)), full((1, H))]),
        out_specs=pl.BlockSpec((EB, H), half),
        out_shape=jax.ShapeDtypeStruct((E, H), jnp.float32),
    )(xji, x, agg_pair,
      w["W_up"], w["Wb0_0"], b("bb0_0"), w["Wb0_1"], b("bb0_1"),
      w["W_skip"], b("b_skip"),
      w["Wa0_0"], b("ba0_0"), w["Wa0_1"], b("ba0_1"),
      w["Wa1_0"], b("ba1_0"), w["Wa1_1"], b("ba1_1"))


def kernel(x, rbf, sbf, idx_kj, idx_ji, w):
    idx_kj = idx_kj.astype(jnp.int32)
    idx_ji = idx_ji.astype(jnp.int32)
    xji, xkj64 = _stage_a(x, rbf, w)
    xgs = _splitter(xkj64)
    se_pair = _stage_b(sbf, w)
    zrows = jnp.zeros((ZR, GW), jnp.float32)
    agg_pair = _stage_c(xgs, se_pair, idx_kj, idx_ji, zrows)
    return _stage_d(xji, x, agg_pair, w)


# speedup vs baseline: 5.9626x; 1.0970x over previous
import functools

import jax
import jax.numpy as jnp
from jax import lax
from jax.experimental import pallas as pl
from jax.experimental.pallas import tpu as pltpu
from jax.experimental.pallas import tpu_sc as plsc

E = 160000
T = 1280000
H = 128
INT = 64
BAS = 8
NR = 6
NSR = 42

NC = 2
NS = 16
L = 16

NG = 8
GW = 8
GPC = NG // NC

TH = T // 2
EH = E // 2
W = 800
PPS = TH // NS
NW = PPS // W
EPS = E // NS
ZR = 500

EB = 2000
EB2 = 2000
TBB = 6400


def _swish(v):
    return v * jax.nn.sigmoid(v)


def _stage_a_body(x_ref, rbf_ref, wji, bji, wkj, bkj, wr12, wdn,
                  xji_out, xkj_out):
    xb = x_ref[...]
    xji = _swish(xb @ wji[...] + bji[...])
    xkj = _swish(xb @ wkj[...] + bkj[...])
    rbf_e = rbf_ref[...] @ wr12[...]
    xkj = xkj * rbf_e
    xkj_out[...] = _swish(xkj @ wdn[...])
    xji_out[...] = xji


def _stage_a(x, rbf, w):
    full = lambda shp: pl.BlockSpec(shp, lambda i: (0, 0))
    wr12 = w["W_rbf1"] @ w["W_rbf2"]
    return pl.pallas_call(
        _stage_a_body,
        grid=(E // EB,),
        in_specs=[pl.BlockSpec((EB, H), lambda i: (i, 0)),
                  pl.BlockSpec((EB, NR), lambda i: (i, 0)),
                  full((H, H)), full((1, H)), full((H, H)), full((1, H)),
                  full((NR, H)), full((H, INT))],
        out_specs=[pl.BlockSpec((EB, H), lambda i: (i, 0)),
                   pl.BlockSpec((EB, INT), lambda i: (i, 0))],
        out_shape=[jax.ShapeDtypeStruct((E, H), jnp.float32),
                   jax.ShapeDtypeStruct((E, INT), jnp.float32)],
    )(x, rbf, w["W_ji"], w["b_ji"].reshape(1, H),
      w["W_kj"], w["b_kj"].reshape(1, H), wr12, w["W_down"])


def _splitter(xkj64):
    def body(xk_ref, *out_refs):
        xk = xk_ref[...]
        for g in range(NG):
            out_refs[g][...] = xk[:, g * GW:(g + 1) * GW]
    return pl.pallas_call(
        body,
        grid=(E // EB2,),
        in_specs=[pl.BlockSpec((EB2, INT), lambda i: (i, 0))],
        out_specs=[pl.BlockSpec((EB2, GW), lambda i: (i, 0))] * NG,
        out_shape=[jax.ShapeDtypeStruct((E, GW), jnp.float32)] * NG,
    )(xkj64)


def _stage_b(sbf, w):
    import numpy as np
    perm = np.zeros((2 * INT, 2 * INT), np.float32)
    for g in range(NG):
        for c in range(GW):
            perm[g * GW + c, 2 * g * GW + c] = 1.0
            perm[INT + g * GW + c, 2 * g * GW + GW + c] = 1.0
    w12 = w["W_sbf1"] @ w["W_sbf2"]
    w12lo = w12 @ jnp.asarray(perm[:INT])
    w12hi = w12 @ jnp.asarray(perm[INT:])

    def body(lo_ref, hi_ref, wlo_ref, whi_ref, out_ref):
        out_ref[...] = lo_ref[...] @ wlo_ref[...] + hi_ref[...] @ whi_ref[...]
    nb = TH // TBB
    return pl.pallas_call(
        body,
        grid=(nb,),
        in_specs=[pl.BlockSpec((TBB, NSR), lambda i: (i, 0)),
                  pl.BlockSpec((TBB, NSR), lambda i, nb=nb: (i + nb, 0)),
                  pl.BlockSpec((NSR, 2 * INT), lambda i: (0, 0)),
                  pl.BlockSpec((NSR, 2 * INT), lambda i: (0, 0))],
        out_specs=pl.BlockSpec((TBB, 2 * INT), lambda i: (i, 0)),
        out_shape=jax.ShapeDtypeStruct((TH, 2 * INT), jnp.float32),
    )(sbf, sbf, w12lo, w12hi)


def _stage_c(xgs, se_pair, idx_kj, idx_ji, zrows):
    mesh = plsc.VectorSubcoreMesh(core_axis_name="c", subcore_axis_name="s")
    out_type = jax.ShapeDtypeStruct((EH, 4 * INT), jnp.float32)
    scratch_types = [
        pltpu.VMEM((2 * W,), jnp.int32),
        pltpu.VMEM((2 * W,), jnp.int32),
        pltpu.VMEM((2 * W, GW), jnp.float32),
        pltpu.VMEM((W, 2 * GW), jnp.float32),
        pltpu.VMEM((2 * W, GW), jnp.float32),
        pltpu.VMEM((ZR, GW), jnp.float32),
        pltpu.VMEM_SHARED((E, GW), jnp.float32),
        pltpu.SemaphoreType.DMA,
    ]

    @functools.partial(pl.kernel, mesh=mesh, out_type=out_type,
                       scratch_types=scratch_types,
                       compiler_params=pltpu.CompilerParams(
                           needs_layout_passes=False,
                           use_tc_tiling_on_sc=False))
    def c_kernel(*refs):
        xg = refs[0:NG]
        se_hbm = refs[NG]
        kj_hbm = refs[NG + 1]
        ji_hbm = refs[NG + 2]
        z_hbm = refs[NG + 3]
        out_hbm = refs[NG + 4]
        kjb, jib, rb, sb, mb, zb, table, sem = refs[NG + 5:]

        cid = lax.axis_index("c")
        sid = lax.axis_index("s")
        base_p = sid * PPS
        base_e = sid * EPS
        out_row = jnp.where(sid < NS // 2, base_e, base_e - EH)
        out_colsel = jnp.where(sid < NS // 2, 0, 2 * INT)

        pltpu.sync_copy(z_hbm, zb)

        lane = lax.iota(jnp.int32, L)
        rsub = jnp.right_shift(lane, 3)
        csub = jnp.bitwise_and(lane, GW - 1)

        def group_pass(g):
            @pl.when(cid == (g // GPC))
            def _():
                for k in range(EPS // ZR):
                    pltpu.sync_copy(zb, table.at[pl.ds(base_e + k * ZR, ZR)])
                plsc.subcore_barrier()

                def win(wi, carry):
                    off = base_p + wi * W
                    d1 = pltpu.async_copy(kj_hbm.at[pl.ds(off, W)],
                                          kjb.at[pl.ds(0, W)], sem)
                    d2 = pltpu.async_copy(kj_hbm.at[pl.ds(TH + off, W)],
                                          kjb.at[pl.ds(W, W)], sem)
                    d3 = pltpu.async_copy(ji_hbm.at[pl.ds(off, W)],
                                          jib.at[pl.ds(0, W)], sem)
                    d4 = pltpu.async_copy(ji_hbm.at[pl.ds(TH + off, W)],
                                          jib.at[pl.ds(W, W)], sem)
                    d5 = pltpu.async_copy(
                        se_hbm.at[pl.ds(off, W), pl.ds(g * 2 * GW, 2 * GW)],
                        sb, sem)
                    d1.wait(); d2.wait(); d3.wait(); d4.wait(); d5.wait()
                    pltpu.async_copy(xg[g].at[kjb], rb, sem).wait()

                    def mul_lo(i, tvec):
                        rv = plsc.load_gather(rb, [tvec, csub])
                        sv = plsc.load_gather(sb, [tvec, csub])
                        plsc.store_scatter(mb, [tvec, csub], rv * sv)
                        return tvec + (L // GW)

                    def mul_hi(i, tvec):
                        rv = plsc.load_gather(rb, [tvec + W, csub])
                        sv = plsc.load_gather(sb, [tvec, csub + GW])
                        plsc.store_scatter(mb, [tvec + W, csub], rv * sv)
                        return tvec + (L // GW)

                    lax.fori_loop(0, W * GW // L, mul_lo, rsub, unroll=10)
                    lax.fori_loop(0, W * GW // L, mul_hi, rsub, unroll=10)
                    pltpu.sync_copy(mb, table.at[jib], add=True)
                    return carry

                lax.fori_loop(0, NW, win, 0)
                plsc.subcore_barrier()
                pltpu.sync_copy(
                    table.at[pl.ds(base_e, EPS)],
                    out_hbm.at[pl.ds(out_row, EPS),
                               pl.ds(out_colsel + g * GW, GW)])
                plsc.subcore_barrier()

        for g in range(NG):
            group_pass(g)

    return c_kernel(*xgs, se_pair, idx_kj, idx_ji, zrows)


def _stage_d_body(xji_ref, x_ref, agg_ref,
                  wup, wb00, bb00, wb01, bb01, wsk, bsk,
                  wa00, ba00, wa01, ba01, wa10, ba10, wa11, ba11, out_ref):
    agg = agg_ref[...][:, :INT]
    xkj = _swish(agg @ wup[...])
    h = xji_ref[...] + xkj
    h = h + _swish(_swish(h @ wb00[...] + bb00[...]) @ wb01[...] + bb01[...])
    h = _swish(h @ wsk[...] + bsk[...]) + x_ref[...]
    h = h + _swish(_swish(h @ wa00[...] + ba00[...]) @ wa01[...] + ba01[...])
    h = h + _swish(_swish(h @ wa10[...] + ba10[...]) @ wa11[...] + ba11[...])
    out_ref[...] = h


def _stage_d(xji, x, agg_pair, w):
    full = lambda shp: pl.BlockSpec(shp, lambda p, i: (0, 0))
    b = lambda name: w[name].reshape(1, H)
    nbe = EH // EB
    half = lambda p, i, nbe=nbe: (p * nbe + i, 0)
    return pl.pallas_call(
        _stage_d_body,
        grid=(2, nbe),
        in_specs=([pl.BlockSpec((EB, H), half),
                   pl.BlockSpec((EB, H), half),
                   pl.BlockSpec((EB, 2 * INT), lambda p, i: (i, p))]
                  + [full((INT, H)),
                     full((H, H)), full((1, H)), full((H, H)), full((1, H)),
                     full((H, H)), full((1, H)),
                     full((H, H)), full((1, H)), full((H, H)), full((1, H)),
                     full((H, H)), full((1, H)), full((H, H)), full((1, H))]),
        out_specs=pl.BlockSpec((EB, H), half),
        out_shape=jax.ShapeDtypeStruct((E, H), jnp.float32),
    )(xji, x, agg_pair,
      w["W_up"], w["Wb0_0"], b("bb0_0"), w["Wb0_1"], b("bb0_1"),
      w["W_skip"], b("b_skip"),
      w["Wa0_0"], b("ba0_0"), w["Wa0_1"], b("ba0_1"),
      w["Wa1_0"], b("ba1_0"), w["Wa1_1"], b("ba1_1"))


def kernel(x, rbf, sbf, idx_kj, idx_ji, w):
    idx_kj = idx_kj.astype(jnp.int32)
    idx_ji = idx_ji.astype(jnp.int32)
    xji, xkj64 = _stage_a(x, rbf, w)
    xgs = _splitter(xkj64)
    se_pair = _stage_b(sbf, w)
    zrows = jnp.zeros((ZR, GW), jnp.float32)
    agg_pair = _stage_c(xgs, se_pair, idx_kj, idx_ji, zrows)
    return _stage_d(xji, x, agg_pair, w)
